# baseline (speedup 1.0000x reference)
import functools

import jax
import jax.numpy as jnp
import numpy as np
from jax import lax
from jax.experimental import pallas as pl
from jax.experimental.pallas import tpu as pltpu
from jax.experimental.pallas import tpu_sc as plsc

MAX_RADIUS = 5.0
AVG_DEG = 32.0
NUM_BASIS = 16

NC = 2
NS = 16
LANES = 16

_COS_PI = (
    9.9999999989e-01, -4.9348021859e+00, 4.0587118172e+00, -1.3352602858e+00,
    2.3532082435e-01, -2.5785806878e-02, 1.9043274683e-03, -8.8690476959e-05,
)


def _coeff_body(r_ref, w1t_ref, b1_ref, w2t_ref, b2_ref, w3_ref, b3_ref,
                out_ref):
    r = r_ref[0]
    inv_step = np.float32((NUM_BASIS - 1) / MAX_RADIUS)
    k_col = lax.broadcasted_iota(
        jnp.int32, (NUM_BASIS, 1), 0).astype(jnp.float32)
    x = r * inv_step - k_col
    u = x * x
    p = jnp.full_like(u, np.float32(_COS_PI[-1]))
    for coef in _COS_PI[-2::-1]:
        p = p * u + np.float32(coef)
    basis = jnp.where(u < 1.0, 0.5 + 0.5 * p, 0.0)
    h = jnp.dot(w1t_ref[...], basis, preferred_element_type=jnp.float32)
    h = jnp.maximum(h + b1_ref[...], 0.0)
    h = jnp.dot(w2t_ref[...], h, preferred_element_type=jnp.float32)
    h = jnp.maximum(h + b2_ref[...], 0.0)
    out = lax.dot_general(h, w3_ref[...], (((0,), (0,)), ((), ())),
                          preferred_element_type=jnp.float32)
    out_ref[...] = out + b3_ref[...]


def _coeff_call(radii2, W1t, b1, W2t, b2, W3, b3, block_e):
    grid, _, BE = radii2.shape
    H, NB = W1t.shape
    D = W3.shape[1]
    assert BE == block_e
    full = lambda shape: pl.BlockSpec(shape, lambda i: (0, 0))
    return pl.pallas_call(
        _coeff_body,
        grid=(grid,),
        in_specs=[
            pl.BlockSpec((1, 1, block_e), lambda i: (i, 0, 0)),
            full((H, NB)),
            full((H, 1)),
            full((H, H)),
            full((H, 1)),
            full((H, D)),
            full((1, D)),
        ],
        out_specs=pl.BlockSpec((block_e, D), lambda i: (i, 0)),
        out_shape=jax.ShapeDtypeStruct((grid * block_e, D), jnp.float32),
    )(radii2, W1t, b1, W2t, b2, W3, b3)


def _sc_edge_body(N_pad, D, C, n_chunks,
                  feat_hbm, coeff_hbm, e_hbm, out_hbm,
                  i0b, i1b, i2b, i3b, rows_v, coeff_v, msg_v,
                  acc_sh, g0, g1, c0, c1, i0s, i1s, i2s, i3s, s_sem):
    idx_b = (i0b, i1b, i2b, i3b)
    i_sems = (i0s, i1s, i2s, i3s)
    g_sems = (g0, g1)
    c_sems = (c0, c1)
    c = lax.axis_index("c")
    s = lax.axis_index("s")
    wid = s * NC + c
    rpt = N_pad // NS

    def zrow(j, carry):
        for k in range(D // LANES):
            msg_v[j, pl.ds(k * LANES, LANES)] = jnp.zeros((LANES,),
                                                          jnp.float32)
        return carry

    lax.fori_loop(0, C, zrow, 0)
    for t in range(rpt // C):
        pltpu.sync_copy(msg_v, acc_sh.at[pl.ds(s * rpt + t * C, C), :])

    def idx_load(j, q):
        pltpu.async_copy(e_hbm.at[:, wid, j], idx_b[q], i_sems[q])

    def idx_wait(j, q):
        pltpu.make_async_copy(e_hbm.at[:, wid, j], idx_b[q],
                              i_sems[q]).wait()

    def load(j, q, b):
        pltpu.async_copy(feat_hbm.at[idx_b[q].at[0]], rows_v.at[b], g_sems[b])
        pltpu.async_copy(coeff_hbm.at[wid, j], coeff_v.at[b], c_sems[b])

    def load_wait(j, q, b):
        pltpu.make_async_copy(feat_hbm.at[idx_b[q].at[0]], rows_v.at[b],
                              g_sems[b]).wait()
        pltpu.make_async_copy(coeff_hbm.at[wid, j], coeff_v.at[b],
                              c_sems[b]).wait()

    idx_load(0, 0)
    idx_load(1, 1)
    idx_wait(0, 0)
    load(0, 0, 0)

    plsc.subcore_barrier()

    def quad(i0, carry):
        for b4 in range(4):
            j = 4 * i0 + b4
            b = b4 % 2
            nb = 1 - b
            q = b4
            nq = (b4 + 1) % 4
            pq = (b4 - 1) % 4

            @pl.when(j < n_chunks)
            def _process():
                @pl.when(j + 1 < n_chunks)
                def _fire_next():
                    idx_wait(j + 1, nq)
                    load(j + 1, nq, nb)

                load_wait(j, q, b)

                @pl.when(j >= 1)
                def _wait_prev_scatter():
                    pltpu.make_async_copy(
                        msg_v, acc_sh.at[idx_b[pq].at[1]], s_sem).wait()

                def mrow(jr, carry2):
                    for k in range(D // LANES):
                        sl = pl.ds(k * LANES, LANES)
                        msg_v[jr, sl] = rows_v[b, jr, sl] * coeff_v[b, jr, sl]
                    return carry2

                lax.fori_loop(0, C, mrow, 0)
                pltpu.async_copy(msg_v, acc_sh.at[idx_b[q].at[1]], s_sem,
                                 add=True)

                @pl.when(j + 2 < n_chunks)
                def _fire_idx():
                    idx_load(j + 2, (b4 + 2) % 4)

        return carry

    lax.fori_loop(0, (n_chunks + 3) // 4, quad, 0)

    pltpu.make_async_copy(msg_v, acc_sh.at[idx_b[(n_chunks - 1) % 4].at[1]],
                          s_sem).wait()

    plsc.subcore_barrier()
    pltpu.sync_copy(acc_sh.at[pl.ds(s * rpt, rpt), :],
                    out_hbm.at[c, pl.ds(s * rpt, rpt), :])


def _sc_call(features, coeff_g, e_s, n_pad, chunk_e):
    N, D = features.shape
    NW, n_chunks, C, _ = coeff_g.shape
    assert C == chunk_e and NW == NC * NS
    mesh = plsc.VectorSubcoreMesh(core_axis_name="c", subcore_axis_name="s")
    body = functools.partial(_sc_edge_body, n_pad, D, C, n_chunks)
    f = pl.kernel(
        body,
        out_type=jax.ShapeDtypeStruct((NC, n_pad, D), jnp.float32),
        mesh=mesh,
        scratch_types=[
            pltpu.VMEM((2, C), jnp.int32),
            pltpu.VMEM((2, C), jnp.int32),
            pltpu.VMEM((2, C), jnp.int32),
            pltpu.VMEM((2, C), jnp.int32),
            pltpu.VMEM((2, C, D), jnp.float32),
            pltpu.VMEM((2, C, D), jnp.float32),
            pltpu.VMEM((C, D), jnp.float32),
            pltpu.VMEM_SHARED((n_pad, D), jnp.float32),
            pltpu.SemaphoreType.DMA,
            pltpu.SemaphoreType.DMA,
            pltpu.SemaphoreType.DMA,
            pltpu.SemaphoreType.DMA,
            pltpu.SemaphoreType.DMA,
            pltpu.SemaphoreType.DMA,
            pltpu.SemaphoreType.DMA,
            pltpu.SemaphoreType.DMA,
            pltpu.SemaphoreType.DMA,
        ],
    )
    return f(features, coeff_g, e_s)


def _gate_body(p0_ref, p1_ref, o_ref):
    a = (p0_ref[0] + p0_ref[1] + p1_ref[0] + p1_ref[1]) * np.float32(
        1.0 / np.sqrt(AVG_DEG))
    o_ref[...] = a * jax.nn.sigmoid(a)


def _gate_call(partial0, partial1, N, block_n):
    _, _, D = partial0.shape
    grid = N // block_n
    spec = pl.BlockSpec((NC, block_n, D), lambda i: (0, i, 0))
    return pl.pallas_call(
        _gate_body,
        grid=(grid,),
        in_specs=[spec, spec],
        out_specs=pl.BlockSpec((block_n, D), lambda i: (i, 0)),
        out_shape=jax.ShapeDtypeStruct((N, D), jnp.float32),
    )(partial0, partial1)


def kernel(features, edge_index, radii, W1, b1, W2, b2, W3, b3):
    N, D = features.shape
    E = radii.shape[0]
    H = W1.shape[1]
    NW = NC * NS

    C = 40
    block_e = 4000
    n_chunks_total = E // (NW * C)
    nc0 = n_chunks_total // 5
    nc1 = n_chunks_total - nc0
    W1t, W2t = W1.T, W2.T
    b1c, b2c = b1.reshape(H, 1), b2.reshape(H, 1)
    b3r = b3.reshape(1, D)

    n_pad = ((N + NS * C - 1) // (NS * C)) * (NS * C)

    partials = []
    ebase = 0
    for nc in (nc0, nc1):
        Es = NW * nc * C
        radii_s = lax.dynamic_slice_in_dim(radii, ebase, Es).reshape(
            Es // block_e, 1, block_e)
        coeff = _coeff_call(radii_s, W1t, b1c, W2t, b2c, W3, b3r,
                            block_e=block_e)
        coeff_g = coeff.reshape(NW, nc, C, D)
        e_s = lax.dynamic_slice_in_dim(edge_index, ebase, Es, axis=1).reshape(
            2, NW, nc, C)
        partials.append(_sc_call(features, coeff_g, e_s,
                                 n_pad=n_pad, chunk_e=C))
        ebase += Es

    return _gate_call(partials[0], partials[1], N, block_n=2000)


# speedup vs baseline: 1.1154x; 1.1154x over previous
import functools

import jax
import jax.numpy as jnp
import numpy as np
from jax import lax
from jax.experimental import pallas as pl
from jax.experimental.pallas import tpu as pltpu
from jax.experimental.pallas import tpu_sc as plsc

MAX_RADIUS = 5.0
AVG_DEG = 32.0
NUM_BASIS = 16

NC = 2
NS = 16
LANES = 16

_COS_PI = (
    9.9999999989e-01, -4.9348021859e+00, 4.0587118172e+00, -1.3352602858e+00,
    2.3532082435e-01, -2.5785806878e-02, 1.9043274683e-03, -8.8690476959e-05,
)


def _coeff_body(r_ref, w1t_ref, b1_ref, w2t_ref, b2_ref, w3_ref, b3_ref,
                out_ref):
    r = r_ref[0]
    inv_step = np.float32((NUM_BASIS - 1) / MAX_RADIUS)
    k_col = lax.broadcasted_iota(
        jnp.int32, (NUM_BASIS, 1), 0).astype(jnp.float32)
    x = r * inv_step - k_col
    u = x * x
    p = jnp.full_like(u, np.float32(_COS_PI[-1]))
    for coef in _COS_PI[-2::-1]:
        p = p * u + np.float32(coef)
    basis = jnp.where(u < 1.0, 0.5 + 0.5 * p, 0.0)
    h = jnp.dot(w1t_ref[...], basis, preferred_element_type=jnp.float32)
    h = jnp.maximum(h + b1_ref[...], 0.0)
    h = jnp.dot(w2t_ref[...], h, preferred_element_type=jnp.float32)
    h = jnp.maximum(h + b2_ref[...], 0.0)
    out = lax.dot_general(h, w3_ref[...], (((0,), (0,)), ((), ())),
                          preferred_element_type=jnp.float32)
    out_ref[...] = out + b3_ref[...]


def _coeff_call(radii2, W1t, b1, W2t, b2, W3, b3, block_e):
    grid, _, BE = radii2.shape
    H, NB = W1t.shape
    D = W3.shape[1]
    assert BE == block_e
    full = lambda shape: pl.BlockSpec(shape, lambda i: (0, 0))
    return pl.pallas_call(
        _coeff_body,
        grid=(grid,),
        in_specs=[
            pl.BlockSpec((1, 1, block_e), lambda i: (i, 0, 0)),
            full((H, NB)),
            full((H, 1)),
            full((H, H)),
            full((H, 1)),
            full((H, D)),
            full((1, D)),
        ],
        out_specs=pl.BlockSpec((block_e, D), lambda i: (i, 0)),
        out_shape=jax.ShapeDtypeStruct((grid * block_e, D), jnp.float32),
    )(radii2, W1t, b1, W2t, b2, W3, b3)


def _sc_edge_body(N_pad, D, C, n_chunks,
                  feat_hbm, coeff_hbm, e_hbm, out_hbm,
                  i0b, i1b, i2b, i3b, rows_v, coeff_v, msg_v,
                  acc_sh, g0, g1, c0, c1, i0s, i1s, i2s, i3s, s_sem):
    idx_b = (i0b, i1b, i2b, i3b)
    i_sems = (i0s, i1s, i2s, i3s)
    g_sems = (g0, g1)
    c_sems = (c0, c1)
    c = lax.axis_index("c")
    s = lax.axis_index("s")
    wid = s * NC + c
    rpt = N_pad // NS

    def zrow(j, carry):
        for k in range(D // LANES):
            msg_v[j, pl.ds(k * LANES, LANES)] = jnp.zeros((LANES,),
                                                          jnp.float32)
        return carry

    lax.fori_loop(0, C, zrow, 0)
    for t in range(rpt // C):
        pltpu.sync_copy(msg_v, acc_sh.at[pl.ds(s * rpt + t * C, C), :])

    def idx_load(j, q):
        pltpu.async_copy(e_hbm.at[:, wid, j], idx_b[q], i_sems[q])

    def idx_wait(j, q):
        pltpu.make_async_copy(e_hbm.at[:, wid, j], idx_b[q],
                              i_sems[q]).wait()

    def load(j, q, b):
        pltpu.async_copy(feat_hbm.at[idx_b[q].at[0]], rows_v.at[b], g_sems[b])
        pltpu.async_copy(coeff_hbm.at[wid, j], coeff_v.at[b], c_sems[b])

    def load_wait(j, q, b):
        pltpu.make_async_copy(feat_hbm.at[idx_b[q].at[0]], rows_v.at[b],
                              g_sems[b]).wait()
        pltpu.make_async_copy(coeff_hbm.at[wid, j], coeff_v.at[b],
                              c_sems[b]).wait()

    idx_load(0, 0)
    idx_load(1, 1)
    idx_wait(0, 0)
    load(0, 0, 0)

    plsc.subcore_barrier()

    def quad(i0, carry):
        for b4 in range(4):
            j = 4 * i0 + b4
            b = b4 % 2
            nb = 1 - b
            q = b4
            nq = (b4 + 1) % 4
            pq = (b4 - 1) % 4

            @pl.when(j < n_chunks)
            def _process():
                @pl.when(j + 1 < n_chunks)
                def _fire_next():
                    idx_wait(j + 1, nq)
                    load(j + 1, nq, nb)

                load_wait(j, q, b)

                @pl.when(j >= 1)
                def _wait_prev_scatter():
                    pltpu.make_async_copy(
                        msg_v, acc_sh.at[idx_b[pq].at[1]], s_sem).wait()

                def mrow(jr, carry2):
                    for k in range(D // LANES):
                        sl = pl.ds(k * LANES, LANES)
                        msg_v[jr, sl] = rows_v[b, jr, sl] * coeff_v[b, jr, sl]
                    return carry2

                lax.fori_loop(0, C, mrow, 0)
                pltpu.async_copy(msg_v, acc_sh.at[idx_b[q].at[1]], s_sem,
                                 add=True)

                @pl.when(j + 2 < n_chunks)
                def _fire_idx():
                    idx_load(j + 2, (b4 + 2) % 4)

        return carry

    lax.fori_loop(0, (n_chunks + 3) // 4, quad, 0)

    pltpu.make_async_copy(msg_v, acc_sh.at[idx_b[(n_chunks - 1) % 4].at[1]],
                          s_sem).wait()

    plsc.subcore_barrier()
    pltpu.sync_copy(acc_sh.at[pl.ds(s * rpt, rpt), :],
                    out_hbm.at[c, pl.ds(s * rpt, rpt), :])


def _sc_call(features, coeff_g, e_s, n_pad, chunk_e):
    N, D = features.shape
    NW, n_chunks, C, _ = coeff_g.shape
    assert C == chunk_e and NW == NC * NS
    mesh = plsc.VectorSubcoreMesh(core_axis_name="c", subcore_axis_name="s")
    body = functools.partial(_sc_edge_body, n_pad, D, C, n_chunks)
    f = pl.kernel(
        body,
        out_type=jax.ShapeDtypeStruct((NC, n_pad, D), jnp.float32),
        mesh=mesh,
        scratch_types=[
            pltpu.VMEM((2, C), jnp.int32),
            pltpu.VMEM((2, C), jnp.int32),
            pltpu.VMEM((2, C), jnp.int32),
            pltpu.VMEM((2, C), jnp.int32),
            pltpu.VMEM((2, C, D), jnp.float32),
            pltpu.VMEM((2, C, D), jnp.float32),
            pltpu.VMEM((C, D), jnp.float32),
            pltpu.VMEM_SHARED((n_pad, D), jnp.float32),
            pltpu.SemaphoreType.DMA,
            pltpu.SemaphoreType.DMA,
            pltpu.SemaphoreType.DMA,
            pltpu.SemaphoreType.DMA,
            pltpu.SemaphoreType.DMA,
            pltpu.SemaphoreType.DMA,
            pltpu.SemaphoreType.DMA,
            pltpu.SemaphoreType.DMA,
            pltpu.SemaphoreType.DMA,
        ],
    )
    return f(features, coeff_g, e_s)


def _gate_body(p0_ref, p1_ref, o_ref):
    a = (p0_ref[0] + p0_ref[1] + p1_ref[0] + p1_ref[1]) * np.float32(
        1.0 / np.sqrt(AVG_DEG))
    o_ref[...] = a * jax.nn.sigmoid(a)


def _gate_call(partial0, partial1, N, block_n):
    _, _, D = partial0.shape
    grid = N // block_n
    spec = pl.BlockSpec((NC, block_n, D), lambda i: (0, i, 0))
    return pl.pallas_call(
        _gate_body,
        grid=(grid,),
        in_specs=[spec, spec],
        out_specs=pl.BlockSpec((block_n, D), lambda i: (i, 0)),
        out_shape=jax.ShapeDtypeStruct((N, D), jnp.float32),
    )(partial0, partial1)


def kernel(features, edge_index, radii, W1, b1, W2, b2, W3, b3):
    N, D = features.shape
    E = radii.shape[0]
    H = W1.shape[1]
    NW = NC * NS

    C = 40
    block_e = 4000
    n_chunks_total = E // (NW * C)
    nc0 = (n_chunks_total * 3) // 10
    nc1 = n_chunks_total - nc0
    W1t, W2t = W1.T, W2.T
    b1c, b2c = b1.reshape(H, 1), b2.reshape(H, 1)
    b3r = b3.reshape(1, D)

    n_pad = ((N + NS * C - 1) // (NS * C)) * (NS * C)

    partials = []
    ebase = 0
    prev_coeff = None
    for nc in (nc0, nc1):
        Es = NW * nc * C
        radii_s = lax.dynamic_slice_in_dim(radii, ebase, Es).reshape(
            Es // block_e, 1, block_e)
        b3s = b3r
        if prev_coeff is not None:
            b3s = b3r + 0.0 * lax.slice(prev_coeff, (0, 0), (1, 1))
        coeff = _coeff_call(radii_s, W1t, b1c, W2t, b2c, W3, b3s,
                            block_e=block_e)
        prev_coeff = coeff
        coeff_g = coeff.reshape(NW, nc, C, D)
        e_s = lax.dynamic_slice_in_dim(edge_index, ebase, Es, axis=1).reshape(
            2, NW, nc, C)
        partials.append(_sc_call(features, coeff_g, e_s,
                                 n_pad=n_pad, chunk_e=C))
        ebase += Es

    return _gate_call(partials[0], partials[1], N, block_n=2000)


# speedup vs baseline: 1.1189x; 1.0032x over previous
import functools

import jax
import jax.numpy as jnp
import numpy as np
from jax import lax
from jax.experimental import pallas as pl
from jax.experimental.pallas import tpu as pltpu
from jax.experimental.pallas import tpu_sc as plsc

MAX_RADIUS = 5.0
AVG_DEG = 32.0
NUM_BASIS = 16

NC = 2
NS = 16
LANES = 16

_COS_PI = (
    9.9999999989e-01, -4.9348021859e+00, 4.0587118172e+00, -1.3352602858e+00,
    2.3532082435e-01, -2.5785806878e-02, 1.9043274683e-03, -8.8690476959e-05,
)


def _coeff_body(r_ref, w1t_ref, b1_ref, w2t_ref, b2_ref, w3_ref, b3_ref,
                out_ref):
    r = r_ref[0]
    inv_step = np.float32((NUM_BASIS - 1) / MAX_RADIUS)
    k_col = lax.broadcasted_iota(
        jnp.int32, (NUM_BASIS, 1), 0).astype(jnp.float32)
    x = r * inv_step - k_col
    u = x * x
    p = jnp.full_like(u, np.float32(_COS_PI[-1]))
    for coef in _COS_PI[-2::-1]:
        p = p * u + np.float32(coef)
    basis = jnp.where(u < 1.0, 0.5 + 0.5 * p, 0.0)
    h = jnp.dot(w1t_ref[...], basis, preferred_element_type=jnp.float32)
    h = jnp.maximum(h + b1_ref[...], 0.0)
    h = jnp.dot(w2t_ref[...], h, preferred_element_type=jnp.float32)
    h = jnp.maximum(h + b2_ref[...], 0.0)
    out = lax.dot_general(h, w3_ref[...], (((0,), (0,)), ((), ())),
                          preferred_element_type=jnp.float32)
    out_ref[...] = out + b3_ref[...]


def _coeff_call(radii2, W1t, b1, W2t, b2, W3, b3, block_e):
    grid, _, BE = radii2.shape
    H, NB = W1t.shape
    D = W3.shape[1]
    assert BE == block_e
    full = lambda shape: pl.BlockSpec(shape, lambda i: (0, 0))
    return pl.pallas_call(
        _coeff_body,
        grid=(grid,),
        in_specs=[
            pl.BlockSpec((1, 1, block_e), lambda i: (i, 0, 0)),
            full((H, NB)),
            full((H, 1)),
            full((H, H)),
            full((H, 1)),
            full((H, D)),
            full((1, D)),
        ],
        out_specs=pl.BlockSpec((block_e, D), lambda i: (i, 0)),
        out_shape=jax.ShapeDtypeStruct((grid * block_e, D), jnp.float32),
    )(radii2, W1t, b1, W2t, b2, W3, b3)


def _sc_edge_body(N_pad, D, C, n_chunks,
                  feat_hbm, coeff_hbm, e_hbm, out_hbm,
                  i0b, i1b, i2b, i3b, rows_v, coeff_v, msg_v,
                  acc_sh, g0, g1, c0, c1, i0s, i1s, i2s, i3s, s_sem):
    idx_b = (i0b, i1b, i2b, i3b)
    i_sems = (i0s, i1s, i2s, i3s)
    g_sems = (g0, g1)
    c_sems = (c0, c1)
    c = lax.axis_index("c")
    s = lax.axis_index("s")
    wid = s * NC + c
    rpt = N_pad // NS

    def zrow(j, carry):
        for k in range(D // LANES):
            msg_v[j, pl.ds(k * LANES, LANES)] = jnp.zeros((LANES,),
                                                          jnp.float32)
        return carry

    lax.fori_loop(0, C, zrow, 0)
    for t in range(rpt // C):
        pltpu.async_copy(msg_v, acc_sh.at[pl.ds(s * rpt + t * C, C), :],
                         s_sem)
    for t in range(rpt // C):
        pltpu.make_async_copy(
            msg_v, acc_sh.at[pl.ds(s * rpt + t * C, C), :], s_sem).wait()

    def idx_load(j, q):
        pltpu.async_copy(e_hbm.at[:, wid, j], idx_b[q], i_sems[q])

    def idx_wait(j, q):
        pltpu.make_async_copy(e_hbm.at[:, wid, j], idx_b[q],
                              i_sems[q]).wait()

    def load(j, q, b):
        pltpu.async_copy(feat_hbm.at[idx_b[q].at[0]], rows_v.at[b], g_sems[b])
        pltpu.async_copy(coeff_hbm.at[wid, j], coeff_v.at[b], c_sems[b])

    def load_wait(j, q, b):
        pltpu.make_async_copy(feat_hbm.at[idx_b[q].at[0]], rows_v.at[b],
                              g_sems[b]).wait()
        pltpu.make_async_copy(coeff_hbm.at[wid, j], coeff_v.at[b],
                              c_sems[b]).wait()

    idx_load(0, 0)
    idx_load(1, 1)
    idx_wait(0, 0)
    load(0, 0, 0)

    plsc.subcore_barrier()

    def quad(i0, carry):
        for b4 in range(4):
            j = 4 * i0 + b4
            b = b4 % 2
            nb = 1 - b
            q = b4
            nq = (b4 + 1) % 4
            pq = (b4 - 1) % 4

            @pl.when(j < n_chunks)
            def _process():
                @pl.when(j + 1 < n_chunks)
                def _fire_next():
                    idx_wait(j + 1, nq)
                    load(j + 1, nq, nb)

                load_wait(j, q, b)

                @pl.when(j >= 1)
                def _wait_prev_scatter():
                    pltpu.make_async_copy(
                        msg_v, acc_sh.at[idx_b[pq].at[1]], s_sem).wait()

                def mrow(jr, carry2):
                    for k in range(D // LANES):
                        sl = pl.ds(k * LANES, LANES)
                        msg_v[jr, sl] = rows_v[b, jr, sl] * coeff_v[b, jr, sl]
                    return carry2

                lax.fori_loop(0, C, mrow, 0)
                pltpu.async_copy(msg_v, acc_sh.at[idx_b[q].at[1]], s_sem,
                                 add=True)

                @pl.when(j + 2 < n_chunks)
                def _fire_idx():
                    idx_load(j + 2, (b4 + 2) % 4)

        return carry

    lax.fori_loop(0, (n_chunks + 3) // 4, quad, 0)

    pltpu.make_async_copy(msg_v, acc_sh.at[idx_b[(n_chunks - 1) % 4].at[1]],
                          s_sem).wait()

    plsc.subcore_barrier()
    pltpu.sync_copy(acc_sh.at[pl.ds(s * rpt, rpt), :],
                    out_hbm.at[c, pl.ds(s * rpt, rpt), :])


def _sc_call(features, coeff_g, e_s, n_pad, chunk_e):
    N, D = features.shape
    NW, n_chunks, C, _ = coeff_g.shape
    assert C == chunk_e and NW == NC * NS
    mesh = plsc.VectorSubcoreMesh(core_axis_name="c", subcore_axis_name="s")
    body = functools.partial(_sc_edge_body, n_pad, D, C, n_chunks)
    f = pl.kernel(
        body,
        out_type=jax.ShapeDtypeStruct((NC, n_pad, D), jnp.float32),
        mesh=mesh,
        scratch_types=[
            pltpu.VMEM((2, C), jnp.int32),
            pltpu.VMEM((2, C), jnp.int32),
            pltpu.VMEM((2, C), jnp.int32),
            pltpu.VMEM((2, C), jnp.int32),
            pltpu.VMEM((2, C, D), jnp.float32),
            pltpu.VMEM((2, C, D), jnp.float32),
            pltpu.VMEM((C, D), jnp.float32),
            pltpu.VMEM_SHARED((n_pad, D), jnp.float32),
            pltpu.SemaphoreType.DMA,
            pltpu.SemaphoreType.DMA,
            pltpu.SemaphoreType.DMA,
            pltpu.SemaphoreType.DMA,
            pltpu.SemaphoreType.DMA,
            pltpu.SemaphoreType.DMA,
            pltpu.SemaphoreType.DMA,
            pltpu.SemaphoreType.DMA,
            pltpu.SemaphoreType.DMA,
        ],
    )
    return f(features, coeff_g, e_s)


def _gate_body(p0_ref, p1_ref, o_ref):
    a = (p0_ref[0] + p0_ref[1] + p1_ref[0] + p1_ref[1]) * np.float32(
        1.0 / np.sqrt(AVG_DEG))
    o_ref[...] = a * jax.nn.sigmoid(a)


def _gate_call(partial0, partial1, N, block_n):
    _, _, D = partial0.shape
    grid = N // block_n
    spec = pl.BlockSpec((NC, block_n, D), lambda i: (0, i, 0))
    return pl.pallas_call(
        _gate_body,
        grid=(grid,),
        in_specs=[spec, spec],
        out_specs=pl.BlockSpec((block_n, D), lambda i: (i, 0)),
        out_shape=jax.ShapeDtypeStruct((N, D), jnp.float32),
    )(partial0, partial1)


def kernel(features, edge_index, radii, W1, b1, W2, b2, W3, b3):
    N, D = features.shape
    E = radii.shape[0]
    H = W1.shape[1]
    NW = NC * NS

    C = 40
    block_e = 4000
    n_chunks_total = E // (NW * C)
    nc0 = (n_chunks_total * 3) // 10
    nc1 = n_chunks_total - nc0
    W1t, W2t = W1.T, W2.T
    b1c, b2c = b1.reshape(H, 1), b2.reshape(H, 1)
    b3r = b3.reshape(1, D)

    n_pad = ((N + NS * C - 1) // (NS * C)) * (NS * C)

    partials = []
    ebase = 0
    prev_coeff = None
    for nc in (nc0, nc1):
        Es = NW * nc * C
        radii_s = lax.dynamic_slice_in_dim(radii, ebase, Es).reshape(
            Es // block_e, 1, block_e)
        b3s = b3r
        if prev_coeff is not None:
            b3s = b3r + 0.0 * lax.slice(prev_coeff, (0, 0), (1, 1))
        coeff = _coeff_call(radii_s, W1t, b1c, W2t, b2c, W3, b3s,
                            block_e=block_e)
        prev_coeff = coeff
        coeff_g = coeff.reshape(NW, nc, C, D)
        e_s = lax.dynamic_slice_in_dim(edge_index, ebase, Es, axis=1).reshape(
            2, NW, nc, C)
        partials.append(_sc_call(features, coeff_g, e_s,
                                 n_pad=n_pad, chunk_e=C))
        ebase += Es

    return _gate_call(partials[0], partials[1], N, block_n=2000)


# speedup vs baseline: 1.1360x; 1.0153x over previous
import functools

import jax
import jax.numpy as jnp
import numpy as np
from jax import lax
from jax.experimental import pallas as pl
from jax.experimental.pallas import tpu as pltpu
from jax.experimental.pallas import tpu_sc as plsc

MAX_RADIUS = 5.0
AVG_DEG = 32.0
NUM_BASIS = 16

NC = 2
NS = 16
LANES = 16

_COS_PI = (
    9.9999999989e-01, -4.9348021859e+00, 4.0587118172e+00, -1.3352602858e+00,
    2.3532082435e-01, -2.5785806878e-02, 1.9043274683e-03, -8.8690476959e-05,
)


def _coeff_body(r_ref, w1t_ref, b1_ref, w2t_ref, b2_ref, w3_ref, b3_ref,
                out_ref):
    r = r_ref[0]
    inv_step = np.float32((NUM_BASIS - 1) / MAX_RADIUS)
    k_col = lax.broadcasted_iota(
        jnp.int32, (NUM_BASIS, 1), 0).astype(jnp.float32)
    x = r * inv_step - k_col
    u = x * x
    p = jnp.full_like(u, np.float32(_COS_PI[-1]))
    for coef in _COS_PI[-2::-1]:
        p = p * u + np.float32(coef)
    basis = jnp.where(u < 1.0, 0.5 + 0.5 * p, 0.0)
    h = jnp.dot(w1t_ref[...], basis, preferred_element_type=jnp.float32)
    h = jnp.maximum(h + b1_ref[...], 0.0)
    h = jnp.dot(w2t_ref[...], h, preferred_element_type=jnp.float32)
    h = jnp.maximum(h + b2_ref[...], 0.0)
    out = lax.dot_general(h, w3_ref[...], (((0,), (0,)), ((), ())),
                          preferred_element_type=jnp.float32)
    out_ref[...] = out + b3_ref[...]


def _coeff_call(radii2, W1t, b1, W2t, b2, W3, b3, block_e):
    grid, _, BE = radii2.shape
    H, NB = W1t.shape
    D = W3.shape[1]
    assert BE == block_e
    full = lambda shape: pl.BlockSpec(shape, lambda i: (0, 0))
    return pl.pallas_call(
        _coeff_body,
        grid=(grid,),
        in_specs=[
            pl.BlockSpec((1, 1, block_e), lambda i: (i, 0, 0)),
            full((H, NB)),
            full((H, 1)),
            full((H, H)),
            full((H, 1)),
            full((H, D)),
            full((1, D)),
        ],
        out_specs=pl.BlockSpec((block_e, D), lambda i: (i, 0)),
        out_shape=jax.ShapeDtypeStruct((grid * block_e, D), jnp.float32),
    )(radii2, W1t, b1, W2t, b2, W3, b3)


def _sc_edge_body(N_pad, D, C, n_chunks,
                  feat_hbm, coeff_hbm, e_hbm, out_hbm,
                  i0b, i1b, i2b, i3b, rows_v, coeff_v, msg_v,
                  acc_sh, g0, g1, c0, c1, i0s, i1s, i2s, i3s, s_sem):
    idx_b = (i0b, i1b, i2b, i3b)
    i_sems = (i0s, i1s, i2s, i3s)
    g_sems = (g0, g1)
    c_sems = (c0, c1)
    c = lax.axis_index("c")
    s = lax.axis_index("s")
    wid = s * NC + c
    rpt = N_pad // NS

    def zrow(j, carry):
        for k in range(D // LANES):
            msg_v[j, pl.ds(k * LANES, LANES)] = jnp.zeros((LANES,),
                                                          jnp.float32)
        return carry

    lax.fori_loop(0, C, zrow, 0)
    for t in range(rpt // C):
        pltpu.async_copy(msg_v, acc_sh.at[pl.ds(s * rpt + t * C, C), :],
                         s_sem)
    for t in range(rpt // C):
        pltpu.make_async_copy(
            msg_v, acc_sh.at[pl.ds(s * rpt + t * C, C), :], s_sem).wait()

    def idx_load(j, q):
        pltpu.async_copy(e_hbm.at[:, wid, j], idx_b[q], i_sems[q])

    def idx_wait(j, q):
        pltpu.make_async_copy(e_hbm.at[:, wid, j], idx_b[q],
                              i_sems[q]).wait()

    def load(j, q, b):
        pltpu.async_copy(feat_hbm.at[idx_b[q].at[0]], rows_v.at[b], g_sems[b])
        pltpu.async_copy(coeff_hbm.at[wid, j], coeff_v.at[b], c_sems[b])

    def load_wait(j, q, b):
        pltpu.make_async_copy(feat_hbm.at[idx_b[q].at[0]], rows_v.at[b],
                              g_sems[b]).wait()
        pltpu.make_async_copy(coeff_hbm.at[wid, j], coeff_v.at[b],
                              c_sems[b]).wait()

    idx_load(0, 0)
    idx_load(1, 1)
    idx_wait(0, 0)
    load(0, 0, 0)

    plsc.subcore_barrier()

    def quad(i0, carry):
        for b4 in range(4):
            j = 4 * i0 + b4
            b = b4 % 2
            nb = 1 - b
            q = b4
            nq = (b4 + 1) % 4
            pq = (b4 - 1) % 4

            @pl.when(j < n_chunks)
            def _process():
                @pl.when(j + 1 < n_chunks)
                def _fire_next():
                    idx_wait(j + 1, nq)
                    load(j + 1, nq, nb)

                load_wait(j, q, b)

                @pl.when(j >= 1)
                def _wait_prev_scatter():
                    pltpu.make_async_copy(
                        msg_v, acc_sh.at[idx_b[pq].at[1]], s_sem).wait()

                def mrow(jr, carry2):
                    for k in range(D // LANES):
                        sl = pl.ds(k * LANES, LANES)
                        msg_v[jr, sl] = rows_v[b, jr, sl] * coeff_v[b, jr, sl]
                    return carry2

                lax.fori_loop(0, C, mrow, 0)
                pltpu.async_copy(msg_v, acc_sh.at[idx_b[q].at[1]], s_sem,
                                 add=True)

                @pl.when(j + 2 < n_chunks)
                def _fire_idx():
                    idx_load(j + 2, (b4 + 2) % 4)

        return carry

    lax.fori_loop(0, (n_chunks + 3) // 4, quad, 0)

    pltpu.make_async_copy(msg_v, acc_sh.at[idx_b[(n_chunks - 1) % 4].at[1]],
                          s_sem).wait()

    plsc.subcore_barrier()
    pltpu.sync_copy(acc_sh.at[pl.ds(s * rpt, rpt), :],
                    out_hbm.at[c, pl.ds(s * rpt, rpt), :])


def _sc_call(features, coeff_g, e_s, n_pad, chunk_e):
    N, D = features.shape
    NW, n_chunks, C, _ = coeff_g.shape
    assert C == chunk_e and NW == NC * NS
    mesh = plsc.VectorSubcoreMesh(core_axis_name="c", subcore_axis_name="s")
    body = functools.partial(_sc_edge_body, n_pad, D, C, n_chunks)
    f = pl.kernel(
        body,
        out_type=jax.ShapeDtypeStruct((NC, n_pad, D), jnp.float32),
        mesh=mesh,
        scratch_types=[
            pltpu.VMEM((2, C), jnp.int32),
            pltpu.VMEM((2, C), jnp.int32),
            pltpu.VMEM((2, C), jnp.int32),
            pltpu.VMEM((2, C), jnp.int32),
            pltpu.VMEM((2, C, D), jnp.float32),
            pltpu.VMEM((2, C, D), jnp.float32),
            pltpu.VMEM((C, D), jnp.float32),
            pltpu.VMEM_SHARED((n_pad, D), jnp.float32),
            pltpu.SemaphoreType.DMA,
            pltpu.SemaphoreType.DMA,
            pltpu.SemaphoreType.DMA,
            pltpu.SemaphoreType.DMA,
            pltpu.SemaphoreType.DMA,
            pltpu.SemaphoreType.DMA,
            pltpu.SemaphoreType.DMA,
            pltpu.SemaphoreType.DMA,
            pltpu.SemaphoreType.DMA,
        ],
    )
    return f(features, coeff_g, e_s)


def _gate_body(p0_ref, p1_ref, o_ref):
    a = (p0_ref[0] + p0_ref[1] + p1_ref[0] + p1_ref[1]) * np.float32(
        1.0 / np.sqrt(AVG_DEG))
    o_ref[...] = a * jax.nn.sigmoid(a)


def _gate_call(partial0, partial1, N, block_n):
    _, _, D = partial0.shape
    grid = N // block_n
    spec = pl.BlockSpec((NC, block_n, D), lambda i: (0, i, 0))
    return pl.pallas_call(
        _gate_body,
        grid=(grid,),
        in_specs=[spec, spec],
        out_specs=pl.BlockSpec((block_n, D), lambda i: (i, 0)),
        out_shape=jax.ShapeDtypeStruct((N, D), jnp.float32),
    )(partial0, partial1)


def kernel(features, edge_index, radii, W1, b1, W2, b2, W3, b3):
    N, D = features.shape
    E = radii.shape[0]
    H = W1.shape[1]
    NW = NC * NS

    C = 40
    n_chunks_total = E // (NW * C)
    nc0 = 60
    nc1 = n_chunks_total - nc0
    W1t, W2t = W1.T, W2.T
    b1c, b2c = b1.reshape(H, 1), b2.reshape(H, 1)
    b3r = b3.reshape(1, D)

    n_pad = ((N + NS * C - 1) // (NS * C)) * (NS * C)

    partials = []
    ebase = 0
    prev_coeff = None
    for nc, block_e in ((nc0, 3840), (nc1, 4864)):
        Es = NW * nc * C
        radii_s = lax.dynamic_slice_in_dim(radii, ebase, Es).reshape(
            Es // block_e, 1, block_e)
        b3s = b3r
        if prev_coeff is not None:
            b3s = b3r + 0.0 * lax.slice(prev_coeff, (0, 0), (1, 1))
        coeff = _coeff_call(radii_s, W1t, b1c, W2t, b2c, W3, b3s,
                            block_e=block_e)
        prev_coeff = coeff
        coeff_g = coeff.reshape(NW, nc, C, D)
        e_s = lax.dynamic_slice_in_dim(edge_index, ebase, Es, axis=1).reshape(
            2, NW, nc, C)
        partials.append(_sc_call(features, coeff_g, e_s,
                                 n_pad=n_pad, chunk_e=C))
        ebase += Es

    return _gate_call(partials[0], partials[1], N, block_n=2000)
